# Initial kernel scaffold; baseline (speedup 1.0000x reference)
import jax
import jax.numpy as jnp
from jax.experimental import pallas as pl


def kernel(x, tables):
    raise NotImplementedError("write your pallas kernel here")


# speedup vs baseline: 21.1914x; 21.1914x over previous
import functools

import jax
import jax.numpy as jnp
import numpy as np
from jax import lax
from jax.experimental import pallas as pl
from jax.experimental.pallas import tpu as pltpu
from jax.experimental.pallas import tpu_sc as plsc

N_LEVELS = 16
F_DIM = 2
HASH_SIZE = 2 ** 19
HASH_MASK = HASH_SIZE - 1
N_POINTS = 262144
_growth = np.exp((np.log(512.0) - np.log(16.0)) / (N_LEVELS - 1))
_RES = [int(np.floor(16.0 * _growth ** l)) for l in range(N_LEVELS)]
_P1 = int(np.int32(np.uint32(2654435761).view(np.int32)))
_P2 = 805459861

NUM_CORES = 2
NUM_SUBCORES = 16
NW = NUM_CORES * NUM_SUBCORES
PTS_PER_W = N_POINTS // NW
C = 1024
NG = C // 16
NCHUNK = PTS_PER_W // C
OUT_W = N_LEVELS * F_DIM


def _encode_body(xt_hbm, tab_hbm, out_hbm,
                 xbuf, lbuf0, lbuf1, idx0, idx1, rows0, rows1, outb,
                 sem0, sem1):
    cid = lax.axis_index("c")
    sid = lax.axis_index("s")
    wid = sid * NUM_CORES + cid
    iota = lax.iota(jnp.int32, 16)

    lbufs = (lbuf0, lbuf1)
    idx_bufs = (idx0, idx1)
    rows_bufs = (rows0, rows1)
    sems = (sem0, sem1)

    def hash_level(l):
        res = float(_RES[l])
        lvl_off = l * HASH_SIZE
        idxb = idx_bufs[l % 2]
        lbuf = lbufs[l % 2]

        def grp(g, carry):
            sl = g * 16
            x0 = xbuf[0, pl.ds(sl, 16)]
            x1 = xbuf[1, pl.ds(sl, 16)]
            x2 = xbuf[2, pl.ds(sl, 16)]
            s0 = x0 * res
            s1 = x1 * res
            s2 = x2 * res
            g0 = s0.astype(jnp.int32)
            g1 = s1.astype(jnp.int32)
            g2 = s2.astype(jnp.int32)
            lbuf[0, pl.ds(sl, 16)] = s0 - g0.astype(jnp.float32)
            lbuf[1, pl.ds(sl, 16)] = s1 - g1.astype(jnp.float32)
            lbuf[2, pl.ds(sl, 16)] = s2 - g2.astype(jnp.float32)
            a1 = g1 * _P1
            a1b = a1 + _P1
            a2 = g2 * _P2
            a2b = a2 + _P2
            g0b = g0 + 1
            g2x = g * 2
            for p in range(8):
                hx = g0b if (p & 1) else g0
                hy = a1b if (p >> 1) & 1 else a1
                hz = a2b if (p >> 2) & 1 else a2
                h = hx ^ hy ^ hz
                e0 = ((h & HASH_MASK) + lvl_off) * 2
                idxb[g2x, pl.ds(16 * p, 16)] = e0
                idxb[g2x + 1, pl.ds(16 * p, 16)] = e0 + 1
            return carry

        lax.fori_loop(jnp.int32(0), jnp.int32(NG), grp, 0, unroll=False)

    def gather_start(l):
        idxb = idx_bufs[l % 2]
        rowsb = rows_bufs[l % 2]
        sem = sems[l % 2]

        def fire(j, carry):
            pltpu.make_async_copy(tab_hbm.at[idxb.at[j]],
                                  rowsb.at[j],
                                  sem).start()
            return carry

        lax.fori_loop(jnp.int32(0), jnp.int32(2 * NG), fire, 0, unroll=False)

    def gather_drain(l):
        idxb = idx_bufs[l % 2]
        rowsb = rows_bufs[l % 2]
        sem = sems[l % 2]

        def drain(j, carry):
            pltpu.make_async_copy(tab_hbm.at[idxb.at[j]],
                                  rowsb.at[j],
                                  sem).wait()
            return carry

        lax.fori_loop(jnp.int32(0), jnp.int32(2 * NG), drain, 0, unroll=False)

    def interp_level(l):
        rowsb = rows_bufs[l % 2]
        lbuf = lbufs[l % 2]

        def grp(g, carry):
            sl = g * 16
            lx = lbuf[0, pl.ds(sl, 16)]
            ly = lbuf[1, pl.ds(sl, 16)]
            lz = lbuf[2, pl.ds(sl, 16)]
            mx = 1.0 - lx
            my = 1.0 - ly
            mz = 1.0 - lz
            pair = (mx * my, mx * ly, lx * my, lx * ly)
            g2x = g * 2
            acc0 = None
            acc1 = None
            for p in range(8):
                a = (p >> 2) & 1
                b = (p >> 1) & 1
                cbit = p & 1
                w = pair[a * 2 + b] * (lz if cbit else mz)
                f0 = rowsb[g2x, pl.ds(16 * p, 16)]
                f1 = rowsb[g2x + 1, pl.ds(16 * p, 16)]
                acc0 = w * f0 if acc0 is None else acc0 + w * f0
                acc1 = w * f1 if acc1 is None else acc1 + w * f1
            outb[2 * l, pl.ds(sl, 16)] = acc0
            outb[2 * l + 1, pl.ds(sl, 16)] = acc1
            return carry

        lax.fori_loop(jnp.int32(0), jnp.int32(NG), grp, 0, unroll=False)

    def chunk(ci, carry):
        base = wid * PTS_PER_W + ci * C
        pltpu.sync_copy(xt_hbm.at[:, pl.ds(base, C)], xbuf)
        hash_level(0)
        gather_start(0)
        for l in range(N_LEVELS):
            if l + 1 < N_LEVELS:
                hash_level(l + 1)
                gather_start(l + 1)
            gather_drain(l)
            interp_level(l)
        pltpu.sync_copy(outb, out_hbm.at[:, pl.ds(base, C)])
        return carry

    lax.fori_loop(jnp.int32(0), jnp.int32(NCHUNK), chunk, 0, unroll=False)


@jax.jit
def kernel(x, tables):
    xt = x.reshape(N_POINTS, 3).T.astype(jnp.float32)
    tab = tables.reshape(N_LEVELS * HASH_SIZE * F_DIM)
    mesh = plsc.VectorSubcoreMesh(core_axis_name="c", subcore_axis_name="s",
                                  num_cores=NUM_CORES,
                                  num_subcores=NUM_SUBCORES)
    enc = pl.kernel(
        _encode_body,
        out_type=jax.ShapeDtypeStruct((OUT_W, N_POINTS), jnp.float32),
        mesh=mesh,
        scratch_types=[
            pltpu.VMEM((3, C), jnp.float32),
            pltpu.VMEM((3, C), jnp.float32),
            pltpu.VMEM((3, C), jnp.float32),
            pltpu.VMEM((2 * NG, 128), jnp.int32),
            pltpu.VMEM((2 * NG, 128), jnp.int32),
            pltpu.VMEM((2 * NG, 128), jnp.float32),
            pltpu.VMEM((2 * NG, 128), jnp.float32),
            pltpu.VMEM((OUT_W, C), jnp.float32),
            pltpu.SemaphoreType.DMA,
            pltpu.SemaphoreType.DMA,
        ],
    )
    return enc(xt, tab).T


# speedup vs baseline: 24.1228x; 1.1383x over previous
import functools

import jax
import jax.numpy as jnp
import numpy as np
from jax import lax
from jax.experimental import pallas as pl
from jax.experimental.pallas import tpu as pltpu
from jax.experimental.pallas import tpu_sc as plsc

N_LEVELS = 16
F_DIM = 2
HASH_SIZE = 2 ** 19
HASH_MASK = HASH_SIZE - 1
N_POINTS = 262144
_growth = np.exp((np.log(512.0) - np.log(16.0)) / (N_LEVELS - 1))
_RES = [int(np.floor(16.0 * _growth ** l)) for l in range(N_LEVELS)]
_P1 = int(np.int32(np.uint32(2654435761).view(np.int32)))
_P2 = 805459861

NUM_CORES = 2
NUM_SUBCORES = 16
NW = NUM_CORES * NUM_SUBCORES
PTS_PER_W = N_POINTS // NW
C = 512
NG = C // 16
NCHUNK = PTS_PER_W // C
OUT_W = N_LEVELS * F_DIM
TAB_E = HASH_SIZE * F_DIM
TAB_SLICE = TAB_E // NUM_SUBCORES


def _encode_body(xt_hbm, tab_hbm, out_hbm,
                 xb0, xb1, lbuf0, lbuf1, idx0, idx1, rows0, rows1, ob0, ob1,
                 tabS, sem0, sem1):
    cid = lax.axis_index("c")
    sid = lax.axis_index("s")
    wid = sid * NUM_CORES + cid
    base_w = wid * PTS_PER_W

    xbufs = (xb0, xb1)
    lbufs = (lbuf0, lbuf1)
    idx_bufs = (idx0, idx1)
    rows_bufs = (rows0, rows1)
    obufs = (ob0, ob1)
    sems = (sem0, sem1)

    def hash_chunk(l, ci, b):
        res = float(_RES[l])
        idxb = idx_bufs[b]
        lbuf = lbufs[b]
        xbuf = xbufs[b]
        pltpu.sync_copy(xt_hbm.at[:, pl.ds(base_w + ci * C, C)], xbuf)

        def grp(g, carry):
            sl = g * 16
            x0 = xbuf[0, pl.ds(sl, 16)]
            x1 = xbuf[1, pl.ds(sl, 16)]
            x2 = xbuf[2, pl.ds(sl, 16)]
            s0 = x0 * res
            s1 = x1 * res
            s2 = x2 * res
            g0 = s0.astype(jnp.int32)
            g1 = s1.astype(jnp.int32)
            g2 = s2.astype(jnp.int32)
            lbuf[0, pl.ds(sl, 16)] = s0 - g0.astype(jnp.float32)
            lbuf[1, pl.ds(sl, 16)] = s1 - g1.astype(jnp.float32)
            lbuf[2, pl.ds(sl, 16)] = s2 - g2.astype(jnp.float32)
            a1 = g1 * _P1
            a1b = a1 + _P1
            a2 = g2 * _P2
            a2b = a2 + _P2
            g0b = g0 + 1
            g2x = g * 2
            for p in range(8):
                hx = g0b if (p & 1) else g0
                hy = a1b if (p >> 1) & 1 else a1
                hz = a2b if (p >> 2) & 1 else a2
                h = hx ^ hy ^ hz
                e0 = (h & HASH_MASK) * 2
                idxb[g2x, pl.ds(16 * p, 16)] = e0
                idxb[g2x + 1, pl.ds(16 * p, 16)] = e0 + 1
            return carry

        lax.fori_loop(jnp.int32(0), jnp.int32(NG), grp, 0, unroll=False)

    def gather_start(b):
        idxb = idx_bufs[b]
        rowsb = rows_bufs[b]
        sem = sems[b]

        def fire(j, carry):
            pltpu.make_async_copy(tabS.at[idxb.at[j]],
                                  rowsb.at[j],
                                  sem).start()
            return carry

        lax.fori_loop(jnp.int32(0), jnp.int32(2 * NG), fire, 0, unroll=False)

    def gather_drain(b):
        idxb = idx_bufs[b]
        rowsb = rows_bufs[b]
        sem = sems[b]

        def drain(j, carry):
            pltpu.make_async_copy(tabS.at[idxb.at[j]],
                                  rowsb.at[j],
                                  sem).wait()
            return carry

        lax.fori_loop(jnp.int32(0), jnp.int32(2 * NG), drain, 0, unroll=False)

    def interp_chunk(l, ci, b):
        rowsb = rows_bufs[b]
        lbuf = lbufs[b]
        ob = obufs[b]

        def grp(g, carry):
            sl = g * 16
            lx = lbuf[0, pl.ds(sl, 16)]
            ly = lbuf[1, pl.ds(sl, 16)]
            lz = lbuf[2, pl.ds(sl, 16)]
            mx = 1.0 - lx
            my = 1.0 - ly
            mz = 1.0 - lz
            pair = (mx * my, mx * ly, lx * my, lx * ly)
            g2x = g * 2
            acc0 = None
            acc1 = None
            for p in range(8):
                a = (p >> 2) & 1
                bb = (p >> 1) & 1
                cbit = p & 1
                w = pair[a * 2 + bb] * (lz if cbit else mz)
                f0 = rowsb[g2x, pl.ds(16 * p, 16)]
                f1 = rowsb[g2x + 1, pl.ds(16 * p, 16)]
                acc0 = w * f0 if acc0 is None else acc0 + w * f0
                acc1 = w * f1 if acc1 is None else acc1 + w * f1
            ob[0, pl.ds(sl, 16)] = acc0
            ob[1, pl.ds(sl, 16)] = acc1
            return carry

        lax.fori_loop(jnp.int32(0), jnp.int32(NG), grp, 0, unroll=False)
        off = pl.multiple_of(l * N_POINTS + base_w + ci * C, C)
        pltpu.sync_copy(ob, out_hbm.at[:, pl.ds(off, C)])

    for l in range(N_LEVELS):
        soff = pl.multiple_of(sid * TAB_SLICE, TAB_SLICE)
        pltpu.sync_copy(tab_hbm.at[pl.ds(l * TAB_E + soff, TAB_SLICE)],
                        tabS.at[pl.ds(soff, TAB_SLICE)])
        plsc.subcore_barrier()

        hash_chunk(l, 0, 0)
        gather_start(0)

        def pairs(j, carry, l=l):
            c0 = j * 2
            hash_chunk(l, c0 + 1, 1)
            gather_start(1)
            gather_drain(0)
            interp_chunk(l, c0, 0)

            @pl.when(j < jnp.int32(NCHUNK // 2 - 1))
            def _():
                hash_chunk(l, c0 + 2, 0)
                gather_start(0)

            gather_drain(1)
            interp_chunk(l, c0 + 1, 1)
            return carry

        lax.fori_loop(jnp.int32(0), jnp.int32(NCHUNK // 2), pairs, 0,
                      unroll=False)
        plsc.subcore_barrier()


@jax.jit
def kernel(x, tables):
    xt = x.reshape(N_POINTS, 3).T.astype(jnp.float32)
    tab = tables.reshape(N_LEVELS * HASH_SIZE * F_DIM)
    mesh = plsc.VectorSubcoreMesh(core_axis_name="c", subcore_axis_name="s",
                                  num_cores=NUM_CORES,
                                  num_subcores=NUM_SUBCORES)
    enc = pl.kernel(
        _encode_body,
        out_type=jax.ShapeDtypeStruct((F_DIM, N_LEVELS * N_POINTS),
                                      jnp.float32),
        mesh=mesh,
        scratch_types=[
            pltpu.VMEM((3, C), jnp.float32),
            pltpu.VMEM((3, C), jnp.float32),
            pltpu.VMEM((3, C), jnp.float32),
            pltpu.VMEM((3, C), jnp.float32),
            pltpu.VMEM((2 * NG, 128), jnp.int32),
            pltpu.VMEM((2 * NG, 128), jnp.int32),
            pltpu.VMEM((2 * NG, 128), jnp.float32),
            pltpu.VMEM((2 * NG, 128), jnp.float32),
            pltpu.VMEM((F_DIM, C), jnp.float32),
            pltpu.VMEM((F_DIM, C), jnp.float32),
            pltpu.VMEM_SHARED((TAB_E,), jnp.float32),
            pltpu.SemaphoreType.DMA,
            pltpu.SemaphoreType.DMA,
        ],
    )
    out = enc(xt, tab)
    return out.reshape(F_DIM, N_LEVELS, N_POINTS).transpose(2, 1, 0) \
              .reshape(N_POINTS, OUT_W)


# speedup vs baseline: 24.5468x; 1.0176x over previous
import functools

import jax
import jax.numpy as jnp
import numpy as np
from jax import lax
from jax.experimental import pallas as pl
from jax.experimental.pallas import tpu as pltpu
from jax.experimental.pallas import tpu_sc as plsc

N_LEVELS = 16
F_DIM = 2
HASH_SIZE = 2 ** 19
HASH_MASK = HASH_SIZE - 1
N_POINTS = 262144
_growth = np.exp((np.log(512.0) - np.log(16.0)) / (N_LEVELS - 1))
_RES = [int(np.floor(16.0 * _growth ** l)) for l in range(N_LEVELS)]
_P1 = int(np.int32(np.uint32(2654435761).view(np.int32)))
_P2 = 805459861

NUM_CORES = 2
NUM_SUBCORES = 16
NW = NUM_CORES * NUM_SUBCORES
PTS_PER_W = N_POINTS // NW
C = 512
NG = C // 16
NCHUNK = PTS_PER_W // C
OUT_W = N_LEVELS * F_DIM
TAB_E = HASH_SIZE * F_DIM
TAB_SLICE = TAB_E // NUM_SUBCORES


def _encode_body(xt_hbm, tab_hbm, out_hbm,
                 xb0, xb1, lbuf0, lbuf1, idx0, idx1, rows0, rows1, ob0, ob1,
                 tabS, sem0, sem1):
    cid = lax.axis_index("c")
    sid = lax.axis_index("s")
    wid = sid * NUM_CORES + cid
    base_w = wid * PTS_PER_W

    xbufs = (xb0, xb1)
    lbufs = (lbuf0, lbuf1)
    idx_bufs = (idx0, idx1)
    rows_bufs = (rows0, rows1)
    obufs = (ob0, ob1)
    sems = (sem0, sem1)

    def hash_chunk(l, ci, b):
        res = float(_RES[l])
        idxb = idx_bufs[b]
        lbuf = lbufs[b]
        xbuf = xbufs[b]
        pltpu.sync_copy(xt_hbm.at[:, pl.ds(base_w + ci * C, C)], xbuf)

        def grp(g, carry):
            sl = g * 16
            x0 = xbuf[0, pl.ds(sl, 16)]
            x1 = xbuf[1, pl.ds(sl, 16)]
            x2 = xbuf[2, pl.ds(sl, 16)]
            s0 = x0 * res
            s1 = x1 * res
            s2 = x2 * res
            g0 = s0.astype(jnp.int32)
            g1 = s1.astype(jnp.int32)
            g2 = s2.astype(jnp.int32)
            lbuf[0, pl.ds(sl, 16)] = s0 - g0.astype(jnp.float32)
            lbuf[1, pl.ds(sl, 16)] = s1 - g1.astype(jnp.float32)
            lbuf[2, pl.ds(sl, 16)] = s2 - g2.astype(jnp.float32)
            a1 = g1 * _P1
            a1b = a1 + _P1
            a2 = g2 * _P2
            a2b = a2 + _P2
            g0b = g0 + 1
            gb = g * 256
            for p in range(8):
                hx = g0b if (p & 1) else g0
                hy = a1b if (p >> 1) & 1 else a1
                hz = a2b if (p >> 2) & 1 else a2
                h = hx ^ hy ^ hz
                e0 = (h & HASH_MASK) * 2
                idxb[pl.ds(gb + 16 * p, 16)] = e0
                idxb[pl.ds(gb + 128 + 16 * p, 16)] = e0 + 1
            return carry

        lax.fori_loop(jnp.int32(0), jnp.int32(NG), grp, 0, unroll=False)

    def gather_start(b):
        pltpu.make_async_copy(tabS.at[idx_bufs[b]],
                              rows_bufs[b],
                              sems[b]).start()

    def gather_drain(b):
        pltpu.make_async_copy(tabS.at[idx_bufs[b]],
                              rows_bufs[b],
                              sems[b]).wait()

    def interp_chunk(l, ci, b):
        rowsb = rows_bufs[b]
        lbuf = lbufs[b]
        ob = obufs[b]

        def grp(g, carry):
            sl = g * 16
            lx = lbuf[0, pl.ds(sl, 16)]
            ly = lbuf[1, pl.ds(sl, 16)]
            lz = lbuf[2, pl.ds(sl, 16)]
            mx = 1.0 - lx
            my = 1.0 - ly
            mz = 1.0 - lz
            pair = (mx * my, mx * ly, lx * my, lx * ly)
            gb = g * 256
            acc0 = None
            acc1 = None
            for p in range(8):
                a = (p >> 2) & 1
                bb = (p >> 1) & 1
                cbit = p & 1
                w = pair[a * 2 + bb] * (lz if cbit else mz)
                f0 = rowsb[pl.ds(gb + 16 * p, 16)]
                f1 = rowsb[pl.ds(gb + 128 + 16 * p, 16)]
                acc0 = w * f0 if acc0 is None else acc0 + w * f0
                acc1 = w * f1 if acc1 is None else acc1 + w * f1
            ob[0, pl.ds(sl, 16)] = acc0
            ob[1, pl.ds(sl, 16)] = acc1
            return carry

        lax.fori_loop(jnp.int32(0), jnp.int32(NG), grp, 0, unroll=False)
        off = pl.multiple_of(l * N_POINTS + base_w + ci * C, C)
        pltpu.sync_copy(ob, out_hbm.at[:, pl.ds(off, C)])

    for l in range(N_LEVELS):
        soff = pl.multiple_of(sid * TAB_SLICE, TAB_SLICE)
        pltpu.sync_copy(tab_hbm.at[pl.ds(l * TAB_E + soff, TAB_SLICE)],
                        tabS.at[pl.ds(soff, TAB_SLICE)])
        plsc.subcore_barrier()

        hash_chunk(l, 0, 0)
        gather_start(0)

        def pairs(j, carry, l=l):
            c0 = j * 2
            hash_chunk(l, c0 + 1, 1)
            gather_start(1)
            gather_drain(0)
            interp_chunk(l, c0, 0)

            @pl.when(j < jnp.int32(NCHUNK // 2 - 1))
            def _():
                hash_chunk(l, c0 + 2, 0)
                gather_start(0)

            gather_drain(1)
            interp_chunk(l, c0 + 1, 1)
            return carry

        lax.fori_loop(jnp.int32(0), jnp.int32(NCHUNK // 2), pairs, 0,
                      unroll=False)
        plsc.subcore_barrier()


@jax.jit
def kernel(x, tables):
    xt = x.reshape(N_POINTS, 3).T.astype(jnp.float32)
    tab = tables.reshape(N_LEVELS * HASH_SIZE * F_DIM)
    mesh = plsc.VectorSubcoreMesh(core_axis_name="c", subcore_axis_name="s",
                                  num_cores=NUM_CORES,
                                  num_subcores=NUM_SUBCORES)
    enc = pl.kernel(
        _encode_body,
        out_type=jax.ShapeDtypeStruct((F_DIM, N_LEVELS * N_POINTS),
                                      jnp.float32),
        mesh=mesh,
        scratch_types=[
            pltpu.VMEM((3, C), jnp.float32),
            pltpu.VMEM((3, C), jnp.float32),
            pltpu.VMEM((3, C), jnp.float32),
            pltpu.VMEM((3, C), jnp.float32),
            pltpu.VMEM((16 * C,), jnp.int32),
            pltpu.VMEM((16 * C,), jnp.int32),
            pltpu.VMEM((16 * C,), jnp.float32),
            pltpu.VMEM((16 * C,), jnp.float32),
            pltpu.VMEM((F_DIM, C), jnp.float32),
            pltpu.VMEM((F_DIM, C), jnp.float32),
            pltpu.VMEM_SHARED((TAB_E,), jnp.float32),
            pltpu.SemaphoreType.DMA,
            pltpu.SemaphoreType.DMA,
        ],
    )
    out = enc(xt, tab)
    return out.reshape(F_DIM, N_LEVELS, N_POINTS).transpose(2, 1, 0) \
              .reshape(N_POINTS, OUT_W)
